# Initial kernel scaffold; baseline (speedup 1.0000x reference)
import jax
import jax.numpy as jnp
from jax.experimental import pallas as pl


def kernel(x, edge_index, W0, b0, W1, b1):
    raise NotImplementedError("write your pallas kernel here")


# speedup vs baseline: 4.3825x; 4.3825x over previous
import functools

import jax
import jax.numpy as jnp
from jax import lax
from jax.experimental import pallas as pl
from jax.experimental.pallas import tpu as pltpu
from jax.experimental.pallas import tpu_sc as plsc

_N = 10000
_E = 320000
_D_IN = 128
_D_HID = 128
_D_OUT = 40
_D_OUT_PAD = 48

_NC = 2
_NS = 16
_NW = _NC * _NS
_C = 128
_KC = 80
_E_PAD = _NW * _KC * _C
_N_PAD = 10240
_ROWS_PER_TILE = _N_PAD // _NS


def _sc_conv(d: int, with_deg: bool):
  mesh = plsc.VectorSubcoreMesh(core_axis_name="c", subcore_axis_name="s")
  if with_deg:
    out_type = (jax.ShapeDtypeStruct((_NC, _N_PAD, d), jnp.float32),
                jax.ShapeDtypeStruct((_NC, _N_PAD), jnp.float32))
  else:
    out_type = jax.ShapeDtypeStruct((_NC, _N_PAD, d), jnp.float32)
  scratch = [
      pltpu.VMEM((_KC, _C), jnp.int32),
      pltpu.VMEM((_KC, _C), jnp.int32),
      pltpu.VMEM((_C, d), jnp.float32),
      pltpu.VMEM((_C,), jnp.float32),
      pltpu.VMEM_SHARED((_N_PAD, d), jnp.float32),
      pltpu.VMEM_SHARED((_N_PAD,), jnp.float32),
      pltpu.SemaphoreType.DMA,
  ]

  def body(h_hbm, sidx_hbm, didx_hbm, zrows_hbm, zdeg_hbm, *rest):
    if with_deg:
      part_hbm, degp_hbm = rest[0], rest[1]
      scr = rest[2:]
    else:
      part_hbm = rest[0]
      scr = rest[1:]
    sidx_v, didx_v, rows_v, ones_v, acc_sh, dacc_sh, sem = scr
    c = lax.axis_index("c")
    s = lax.axis_index("s")
    wid = c * _NS + s
    row0 = s * _ROWS_PER_TILE

    pltpu.sync_copy(zrows_hbm.at[pl.ds(row0, _ROWS_PER_TILE)],
                    acc_sh.at[pl.ds(row0, _ROWS_PER_TILE)])
    if with_deg:
      pltpu.sync_copy(zdeg_hbm.at[pl.ds(row0, _ROWS_PER_TILE)],
                      dacc_sh.at[pl.ds(row0, _ROWS_PER_TILE)])
      for i in range(_C // 16):
        ones_v[pl.ds(16 * i, 16)] = jnp.full((16,), 1.0, jnp.float32)
    pltpu.sync_copy(sidx_hbm.at[wid], sidx_v)
    pltpu.sync_copy(didx_hbm.at[wid], didx_v)
    plsc.subcore_barrier()

    def chunk(j, carry):
      pltpu.async_copy(h_hbm.at[sidx_v.at[j]], rows_v, sem).wait()
      pltpu.sync_copy(rows_v, acc_sh.at[didx_v.at[j]], add=True)
      if with_deg:
        pltpu.sync_copy(ones_v, dacc_sh.at[didx_v.at[j]], add=True)
      return carry

    lax.fori_loop(0, _KC, chunk, 0)
    plsc.subcore_barrier()

    pltpu.sync_copy(acc_sh.at[pl.ds(row0, _ROWS_PER_TILE)],
                    part_hbm.at[c, pl.ds(row0, _ROWS_PER_TILE)])
    if with_deg:
      pltpu.sync_copy(dacc_sh.at[pl.ds(row0, _ROWS_PER_TILE)],
                      degp_hbm.at[c, pl.ds(row0, _ROWS_PER_TILE)])

  params = None
  if d % 128 != 0:
    params = pltpu.CompilerParams(use_tc_tiling_on_sc=False)
  return pl.kernel(body, out_type=out_type, mesh=mesh, scratch_types=scratch,
                   compiler_params=params, name=f"sc_conv_d{d}")


def _erf(z):
  a1, a2, a3, a4, a5 = (0.254829592, -0.284496736, 1.421413741,
                        -1.453152027, 1.061405429)
  p = 0.3275911
  az = jnp.abs(z)
  t = 1.0 / (1.0 + p * az)
  poly = t * (a1 + t * (a2 + t * (a3 + t * (a4 + t * a5))))
  e = 1.0 - poly * jnp.exp(-az * az)
  return jnp.sign(z) * e


def _mm_body(x_ref, w_ref, o_ref):
  o_ref[...] = jnp.dot(x_ref[...], w_ref[...],
                       preferred_element_type=jnp.float32)


def _mid_body(part_ref, degp_ref, b0_ref, w1_ref, o_ref):
  deg = jnp.maximum(degp_ref[0] + degp_ref[1], 1.0)
  h = (part_ref[0] + part_ref[1]) / deg + b0_ref[...]
  m = jnp.mean(h, axis=-1, keepdims=True)
  hc = h - m
  v = jnp.mean(hc * hc, axis=-1, keepdims=True)
  hn = hc / jnp.sqrt(v + 1e-5)
  g = 0.5 * hn * (1.0 + _erf(hn * 0.7071067811865476))
  o_ref[...] = jnp.dot(g, w1_ref[...], preferred_element_type=jnp.float32)


def _out_body(part_ref, degp_ref, b1_ref, o_ref):
  deg = jnp.maximum(degp_ref[0] + degp_ref[1], 1.0)
  t = (part_ref[0] + part_ref[1]) / deg + b1_ref[...]
  col = lax.broadcasted_iota(jnp.int32, t.shape, 1)
  t = jnp.where(col < _D_OUT, t, -1e30)
  mx = jnp.max(t, axis=-1, keepdims=True)
  lse = mx + jnp.log(jnp.sum(jnp.exp(t - mx), axis=-1, keepdims=True))
  o_ref[...] = t - lse


_RB = 512
_G = _N_PAD // _RB


def _tc_matmul(x, w):
  return pl.pallas_call(
      _mm_body,
      grid=(_G,),
      in_specs=[pl.BlockSpec((_RB, _D_IN), lambda i: (i, 0)),
                pl.BlockSpec((_D_IN, _D_HID), lambda i: (0, 0))],
      out_specs=pl.BlockSpec((_RB, _D_HID), lambda i: (i, 0)),
      out_shape=jax.ShapeDtypeStruct((_N_PAD, _D_HID), jnp.float32),
  )(x, w)


def _tc_mid(part, degp, b0, w1p):
  return pl.pallas_call(
      _mid_body,
      grid=(_G,),
      in_specs=[pl.BlockSpec((_NC, _RB, _D_HID), lambda i: (0, i, 0)),
                pl.BlockSpec((_NC, _RB, 1), lambda i: (0, i, 0)),
                pl.BlockSpec((1, _D_HID), lambda i: (0, 0)),
                pl.BlockSpec((_D_HID, _D_OUT_PAD), lambda i: (0, 0))],
      out_specs=pl.BlockSpec((_RB, _D_OUT_PAD), lambda i: (i, 0)),
      out_shape=jax.ShapeDtypeStruct((_N_PAD, _D_OUT_PAD), jnp.float32),
  )(part, degp, b0, w1p)


def _tc_out(part, degp, b1p):
  return pl.pallas_call(
      _out_body,
      grid=(_G,),
      in_specs=[pl.BlockSpec((_NC, _RB, _D_OUT_PAD), lambda i: (0, i, 0)),
                pl.BlockSpec((_NC, _RB, 1), lambda i: (0, i, 0)),
                pl.BlockSpec((1, _D_OUT_PAD), lambda i: (0, 0))],
      out_specs=pl.BlockSpec((_RB, _D_OUT_PAD), lambda i: (i, 0)),
      out_shape=jax.ShapeDtypeStruct((_N_PAD, _D_OUT_PAD), jnp.float32),
  )(part, degp, b1p)


@jax.jit
def kernel(x, edge_index, W0, b0, W1, b1):
  src = edge_index[0]
  dst = edge_index[1]
  pad = _E_PAD - _E
  src_p = jnp.concatenate([src, jnp.zeros((pad,), jnp.int32)])
  dst_p = jnp.concatenate([dst, jnp.full((pad,), _N, jnp.int32)])
  sidx = src_p.reshape(_NW, _KC, _C)
  didx = dst_p.reshape(_NW, _KC, _C)

  x_pad = jnp.zeros((_N_PAD, _D_IN), jnp.float32).at[:_N].set(x)
  w1p = jnp.zeros((_D_HID, _D_OUT_PAD), jnp.float32).at[:, :_D_OUT].set(W1)
  b1p = jnp.zeros((1, _D_OUT_PAD), jnp.float32).at[0, :_D_OUT].set(b1)
  b0r = b0.reshape(1, _D_HID)

  zrows = jnp.zeros((_N_PAD, _D_HID), jnp.float32)
  zrows2 = jnp.zeros((_N_PAD, _D_OUT_PAD), jnp.float32)
  zdeg = jnp.zeros((_N_PAD,), jnp.float32)

  h1 = _tc_matmul(x_pad, W0)
  part1, degp = _sc_conv(_D_HID, True)(h1, sidx, didx, zrows, zdeg)
  degp3 = degp.reshape(_NC, _N_PAD, 1)
  h2 = _tc_mid(part1, degp3, b0r, w1p)
  part2 = _sc_conv(_D_OUT_PAD, False)(h2, sidx, didx, zrows2, zdeg)
  out = _tc_out(part2, degp3, b1p)
  return out[:_N, :_D_OUT]


# speedup vs baseline: 5.9393x; 1.3552x over previous
import jax
import jax.numpy as jnp
from jax import lax
from jax.experimental import pallas as pl
from jax.experimental.pallas import tpu as pltpu
from jax.experimental.pallas import tpu_sc as plsc

_N = 10000
_E = 320000
_D_IN = 128
_D_HID = 128
_D_HALF = 64
_D_OUT = 40
_D_OUT_PAD = 48

_NC = 2
_NS = 16
_NW = _NC * _NS
_C = 128
_NCH = 2560
_E_PAD = _NCH * _C
_N_PAD = 10240
_ROWS_PER_TILE = _N_PAD // _NS
_NBUF = 4


def _sc_conv(d: int, col_split: bool, with_deg: bool):
  mesh = plsc.VectorSubcoreMesh(core_axis_name="c", subcore_axis_name="s")
  kc = _NCH // _NS if col_split else _NCH // _NW
  if with_deg:
    out_type = (jax.ShapeDtypeStruct((_NC, _N_PAD, d), jnp.float32),
                jax.ShapeDtypeStruct((_NC, _N_PAD), jnp.float32))
  else:
    out_type = jax.ShapeDtypeStruct((_NC, _N_PAD, d), jnp.float32)
  scratch = [
      pltpu.VMEM((kc, _C), jnp.int32),
      pltpu.VMEM((kc, _C), jnp.int32),
      pltpu.VMEM((_NBUF, _C, d), jnp.float32),
      pltpu.VMEM((_C,), jnp.float32),
      pltpu.VMEM_SHARED((_N_PAD, d), jnp.float32),
      pltpu.VMEM_SHARED((_N_PAD,), jnp.float32),
  ] + [pltpu.SemaphoreType.DMA] * _NBUF

  def body(h_hbm, sidx_hbm, didx_hbm, zrows_hbm, zdeg_hbm, *rest):
    if with_deg:
      part_hbm, degp_hbm = rest[0], rest[1]
      scr = rest[2:]
    else:
      part_hbm = rest[0]
      scr = rest[1:]
    sidx_v, didx_v, rows_v, ones_v, acc_sh, dacc_sh = scr[:6]
    sems = scr[6:6 + _NBUF]
    c = lax.axis_index("c")
    s = lax.axis_index("s")
    row0 = s * _ROWS_PER_TILE

    pltpu.sync_copy(zrows_hbm.at[pl.ds(row0, _ROWS_PER_TILE)],
                    acc_sh.at[pl.ds(row0, _ROWS_PER_TILE)])
    if with_deg:
      pltpu.sync_copy(zdeg_hbm.at[pl.ds(row0, _ROWS_PER_TILE)],
                      dacc_sh.at[pl.ds(row0, _ROWS_PER_TILE)])
      for i in range(_C // 16):
        ones_v[pl.ds(16 * i, 16)] = jnp.full((16,), 1.0, jnp.float32)
    if col_split:
      chunk0 = s * kc
      pltpu.sync_copy(sidx_hbm.at[c, pl.ds(chunk0, kc)], sidx_v)
      pltpu.sync_copy(didx_hbm.at[pl.ds(chunk0, kc)], didx_v)
    else:
      chunk0 = (c * _NS + s) * kc
      pltpu.sync_copy(sidx_hbm.at[pl.ds(chunk0, kc)], sidx_v)
      pltpu.sync_copy(didx_hbm.at[pl.ds(chunk0, kc)], didx_v)
    plsc.subcore_barrier()

    def consume(j, b):
      pltpu.make_async_copy(h_hbm.at[sidx_v.at[j]], rows_v.at[b],
                            sems[b]).wait()
      pltpu.sync_copy(rows_v.at[b], acc_sh.at[didx_v.at[j]], add=True)
      if with_deg:
        pltpu.sync_copy(ones_v, dacc_sh.at[didx_v.at[j]], add=True)

    for b in range(_NBUF):
      pltpu.async_copy(h_hbm.at[sidx_v.at[b]], rows_v.at[b], sems[b])

    def group(g, carry):
      for b in range(_NBUF):
        j = g * _NBUF + b
        consume(j, b)
        pltpu.async_copy(h_hbm.at[sidx_v.at[j + _NBUF]], rows_v.at[b], sems[b])
      return carry

    lax.fori_loop(0, kc // _NBUF - 1, group, 0)
    for b in range(_NBUF):
      consume(kc - _NBUF + b, b)
    plsc.subcore_barrier()

    pltpu.sync_copy(acc_sh.at[pl.ds(row0, _ROWS_PER_TILE)],
                    part_hbm.at[c, pl.ds(row0, _ROWS_PER_TILE)])
    if with_deg:
      pltpu.sync_copy(dacc_sh.at[pl.ds(row0, _ROWS_PER_TILE)],
                      degp_hbm.at[c, pl.ds(row0, _ROWS_PER_TILE)])

  params = None
  if d % 128 != 0:
    params = pltpu.CompilerParams(use_tc_tiling_on_sc=False)
  return pl.kernel(body, out_type=out_type, mesh=mesh, scratch_types=scratch,
                   compiler_params=params, name=f"sc_conv_d{d}")


def _erf(z):
  a1, a2, a3, a4, a5 = (0.254829592, -0.284496736, 1.421413741,
                        -1.453152027, 1.061405429)
  p = 0.3275911
  az = jnp.abs(z)
  t = 1.0 / (1.0 + p * az)
  poly = t * (a1 + t * (a2 + t * (a3 + t * (a4 + t * a5))))
  e = 1.0 - poly * jnp.exp(-az * az)
  return jnp.sign(z) * e


def _mm_body(x_ref, w_ref, o_ref):
  o_ref[0] = jnp.dot(x_ref[...], w_ref[0],
                     preferred_element_type=jnp.float32)


def _mid_body(part_ref, deg_ref, b0_ref, w1_ref, o_ref):
  deg = jnp.maximum(deg_ref[...], 1.0)
  agg = jnp.concatenate([part_ref[0], part_ref[1]], axis=-1)
  h = agg / deg + b0_ref[...]
  m = jnp.mean(h, axis=-1, keepdims=True)
  hc = h - m
  v = jnp.mean(hc * hc, axis=-1, keepdims=True)
  hn = hc / jnp.sqrt(v + 1e-5)
  g = 0.5 * hn * (1.0 + _erf(hn * 0.7071067811865476))
  o_ref[...] = jnp.dot(g, w1_ref[...], preferred_element_type=jnp.float32)


def _out_body(part_ref, deg_ref, b1_ref, o_ref):
  deg = jnp.maximum(deg_ref[...], 1.0)
  t = (part_ref[0] + part_ref[1]) / deg + b1_ref[...]
  col = lax.broadcasted_iota(jnp.int32, t.shape, 1)
  t = jnp.where(col < _D_OUT, t, -1e30)
  mx = jnp.max(t, axis=-1, keepdims=True)
  lse = mx + jnp.log(jnp.sum(jnp.exp(t - mx), axis=-1, keepdims=True))
  o_ref[...] = t - lse


_RB = 512
_G = _N_PAD // _RB


def _tc_matmul(x, w):
  return pl.pallas_call(
      _mm_body,
      grid=(_NC, _G),
      in_specs=[pl.BlockSpec((_RB, _D_IN), lambda h, i: (i, 0)),
                pl.BlockSpec((1, _D_IN, _D_HALF), lambda h, i: (h, 0, 0))],
      out_specs=pl.BlockSpec((1, _RB, _D_HALF), lambda h, i: (h, i, 0)),
      out_shape=jax.ShapeDtypeStruct((_NC, _N_PAD, _D_HALF), jnp.float32),
  )(x, w)


def _tc_mid(part, deg, b0, w1p):
  return pl.pallas_call(
      _mid_body,
      grid=(_G,),
      in_specs=[pl.BlockSpec((_NC, _RB, _D_HALF), lambda i: (0, i, 0)),
                pl.BlockSpec((_RB, 1), lambda i: (i, 0)),
                pl.BlockSpec((1, _D_HID), lambda i: (0, 0)),
                pl.BlockSpec((_D_HID, _D_OUT_PAD), lambda i: (0, 0))],
      out_specs=pl.BlockSpec((_RB, _D_OUT_PAD), lambda i: (i, 0)),
      out_shape=jax.ShapeDtypeStruct((_N_PAD, _D_OUT_PAD), jnp.float32),
  )(part, deg, b0, w1p)


def _tc_out(part, deg, b1p):
  return pl.pallas_call(
      _out_body,
      grid=(_G,),
      in_specs=[pl.BlockSpec((_NC, _RB, _D_OUT_PAD), lambda i: (0, i, 0)),
                pl.BlockSpec((_RB, 1), lambda i: (i, 0)),
                pl.BlockSpec((1, _D_OUT_PAD), lambda i: (0, 0))],
      out_specs=pl.BlockSpec((_RB, _D_OUT_PAD), lambda i: (i, 0)),
      out_shape=jax.ShapeDtypeStruct((_N_PAD, _D_OUT_PAD), jnp.float32),
  )(part, deg, b1p)


@jax.jit
def kernel(x, edge_index, W0, b0, W1, b1):
  src = edge_index[0]
  dst = edge_index[1]
  pad = _E_PAD - _E
  src_p = jnp.concatenate([src, jnp.zeros((pad,), jnp.int32)])
  dst_p = jnp.concatenate([dst, jnp.full((pad,), _N, jnp.int32)])
  sidx = src_p.reshape(_NCH, _C)
  didx = dst_p.reshape(_NCH, _C)
  sidx2 = jnp.stack([sidx, sidx + _N_PAD])

  x_pad = jnp.zeros((_N_PAD, _D_IN), jnp.float32).at[:_N].set(x)
  w1p = jnp.zeros((_D_HID, _D_OUT_PAD), jnp.float32).at[:, :_D_OUT].set(W1)
  b1p = jnp.zeros((1, _D_OUT_PAD), jnp.float32).at[0, :_D_OUT].set(b1)
  b0r = b0.reshape(1, _D_HID)

  zrows = jnp.zeros((_N_PAD, _D_HALF), jnp.float32)
  zrows2 = jnp.zeros((_N_PAD, _D_OUT_PAD), jnp.float32)
  zdeg = jnp.zeros((_N_PAD,), jnp.float32)

  w0h = jnp.stack([W0[:, :_D_HALF], W0[:, _D_HALF:]])
  h1 = _tc_matmul(x_pad, w0h)
  h1f = h1.reshape(_NC * _N_PAD, _D_HALF)
  part1, degp = _sc_conv(_D_HALF, True, True)(h1f, sidx2, didx, zrows, zdeg)
  deg = degp[0].reshape(_N_PAD, 1)
  h2 = _tc_mid(part1, deg, b0r, w1p)
  part2 = _sc_conv(_D_OUT_PAD, False, False)(h2, sidx, didx, zrows2, zdeg)
  out = _tc_out(part2, deg, b1p)
  return out[:_N, :_D_OUT]
